# baseline (speedup 1.0000x reference)
import functools

import jax
import jax.numpy as jnp
from jax import lax
from jax.experimental import pallas as pl
from jax.experimental.pallas import tpu as pltpu
from jax.experimental.pallas import tpu_sc as plsc

NC, NS = 2, 16
NW = NC * NS
CH = 100
L = 16
NB = 2

_TAKE_DNUMS = lax.GatherDimensionNumbers(
    offset_dims=(), collapsed_slice_dims=(0,), start_index_map=(0,))


def _permute(t, idx):
    return lax.gather(t, idx[:, None], _TAKE_DNUMS, slice_sizes=(1,),
                      mode=lax.GatherScatterMode.PROMISE_IN_BOUNDS)


def _sc_gemv(embed, idx2d, weight, bias1d, B, S, D):
    EPW = B // NW
    CPW = (B * S) // (CH * NW)
    CPE = S // CH
    KD = D // L

    mesh = plsc.VectorSubcoreMesh(core_axis_name="c", subcore_axis_name="s")

    @functools.partial(
        pl.kernel,
        out_type=jax.ShapeDtypeStruct((B, CPE, CH), jnp.float32),
        mesh=mesh,
        scratch_types=[
            pltpu.VMEM((CPW, CH), jnp.int32),
            pltpu.VMEM((EPW, D), jnp.float32),
            pltpu.VMEM((NB, CH, D), jnp.float32),
            pltpu.VMEM((NB, CH), jnp.float32),
            pltpu.VMEM((EPW, CPE, CH), jnp.float32),
            pltpu.SemaphoreType.DMA,
            pltpu.SemaphoreType.DMA,
            pltpu.SemaphoreType.DMA,
            pltpu.SemaphoreType.DMA,
        ],
    )
    def run(emb_hbm, idx_hbm, w_hbm, b_hbm, out_hbm,
            idx_v, emb_v, rows_v, biasg_v, out_v,
            sem_r0, sem_r1, sem_b0, sem_b1):
        wid = lax.axis_index("s") * NC + lax.axis_index("c")
        sems_r = (sem_r0, sem_r1)
        sems_b = (sem_b0, sem_b1)

        pltpu.sync_copy(idx_hbm.at[pl.ds(wid * CPW, CPW)], idx_v)
        pltpu.sync_copy(emb_hbm.at[pl.ds(wid * EPW, EPW)], emb_v)

        def rows_copy(c, slot):
            return pltpu.make_async_copy(
                w_hbm.at[idx_v.at[c]], rows_v.at[slot], sems_r[slot])

        def bias_copy(c, slot):
            return pltpu.make_async_copy(
                b_hbm.at[idx_v.at[c]], biasg_v.at[slot], sems_b[slot])

        def start(c, slot):
            rows_copy(c, slot).start()
            bias_copy(c, slot).start()

        lanes = lax.iota(jnp.int32, L)

        def compute(c, slot):
            e = c // CPE
            h = c % CPE
            ev = [emb_v[e, pl.ds(L * k, L)] for k in range(KD)]

            def group(off):
                ts = []
                for j in range(L):
                    s = off + j
                    acc = rows_v[slot, s, pl.ds(0, L)] * ev[0]
                    for k in range(1, KD):
                        acc = acc + rows_v[slot, s, pl.ds(L * k, L)] * ev[k]
                    ts.append(acc)
                d = 1
                while len(ts) > 1:
                    m = (lanes & d) == 0
                    ts = [jnp.where(m, a, b) +
                          _permute(jnp.where(m, b, a), lanes ^ d)
                          for a, b in zip(ts[0::2], ts[1::2])]
                    d *= 2
                out_v[e, h, pl.ds(off, L)] = ts[0] + biasg_v[slot, pl.ds(off, L)]

            def gbody(g, carry):
                group(g * L)
                return carry

            lax.fori_loop(0, CH // L, gbody, 0)
            group(CH - L)

        for b in range(NB):
            start(b, b)

        def outer(g, carry):
            for b in range(NB):
                c = NB * g + b
                rows_copy(c, b).wait()
                bias_copy(c, b).wait()
                compute(c, b)

                @pl.when(c + NB < CPW)
                def _():
                    start(c + NB, b)
            return carry

        lax.fori_loop(0, CPW // NB, outer, 0)

        pltpu.sync_copy(out_v, out_hbm.at[pl.ds(wid * EPW, EPW)])

    return run(embed, idx2d, weight, bias1d)


def kernel(embed, shortlist, weight, bias):
    B, D = embed.shape
    S = shortlist.shape[1]
    idx2d = shortlist.astype(jnp.int32).reshape(B * S // CH, CH)
    bias1d = bias.T.reshape(bias.shape[0])
    return _sc_gemv(embed, idx2d, weight, bias1d, B, S, D).reshape(B, S)


# speedup vs baseline: 1.0231x; 1.0231x over previous
import functools

import jax
import jax.numpy as jnp
from jax import lax
from jax.experimental import pallas as pl
from jax.experimental.pallas import tpu as pltpu
from jax.experimental.pallas import tpu_sc as plsc

NC, NS = 2, 16
NW = NC * NS
CH = 100
L = 16
NB = 2

_TAKE_DNUMS = lax.GatherDimensionNumbers(
    offset_dims=(), collapsed_slice_dims=(0,), start_index_map=(0,))


def _permute(t, idx):
    return lax.gather(t, idx[:, None], _TAKE_DNUMS, slice_sizes=(1,),
                      mode=lax.GatherScatterMode.PROMISE_IN_BOUNDS)


def _sc_gemv(embed, idx2d, weight, bias1d, B, S, D):
    EPW = B // NW
    CPW = (B * S) // (CH * NW)
    CPE = S // CH
    KD = D // L

    mesh = plsc.VectorSubcoreMesh(core_axis_name="c", subcore_axis_name="s")

    @functools.partial(
        pl.kernel,
        out_type=jax.ShapeDtypeStruct((B, CPE, CH), jnp.float32),
        mesh=mesh,
        scratch_types=[
            pltpu.VMEM((CPW, CH), jnp.int32),
            pltpu.VMEM((EPW, D), jnp.float32),
            pltpu.VMEM((NB, CH, D), jnp.float32),
            pltpu.VMEM((NB, CH), jnp.float32),
            pltpu.VMEM((EPW, CPE, CH), jnp.float32),
            pltpu.SemaphoreType.DMA,
            pltpu.SemaphoreType.DMA,
            pltpu.SemaphoreType.DMA,
            pltpu.SemaphoreType.DMA,
        ],
    )
    def run(emb_hbm, idx_hbm, w_hbm, b_hbm, out_hbm,
            idx_v, emb_v, rows_v, biasg_v, out_v,
            sem_r0, sem_r1, sem_b0, sem_b1):
        wid = lax.axis_index("s") * NC + lax.axis_index("c")
        sems_r = (sem_r0, sem_r1)
        sems_b = (sem_b0, sem_b1)

        pltpu.sync_copy(idx_hbm.at[pl.ds(wid * CPW, CPW)], idx_v)
        pltpu.sync_copy(emb_hbm.at[pl.ds(wid * EPW, EPW)], emb_v)

        def rows_copy(c, slot):
            return pltpu.make_async_copy(
                w_hbm.at[idx_v.at[c]], rows_v.at[slot], sems_r[slot])

        def bias_copy(c, slot):
            return pltpu.make_async_copy(
                b_hbm.at[idx_v.at[c]], biasg_v.at[slot], sems_b[slot])

        def start(c, slot):
            rows_copy(c, slot).start()
            bias_copy(c, slot).start()

        lanes = lax.iota(jnp.int32, L)

        def compute(c, slot):
            e = c // CPE
            h = c % CPE
            ev = [emb_v[e, pl.ds(L * k, L)] for k in range(KD)]

            def entry_dot(s):
                acc = rows_v[slot, s, pl.ds(0, L)] * ev[0]
                for k in range(1, KD):
                    acc = acc + rows_v[slot, s, pl.ds(L * k, L)] * ev[k]
                return acc

            def combine(ts, d):
                while len(ts) > 1:
                    m = (lanes & d) == 0
                    ts = [jnp.where(m, a, b) +
                          _permute(jnp.where(m, b, a), lanes ^ d)
                          for a, b in zip(ts[0::2], ts[1::2])]
                    d *= 2
                return ts[0], d

            def group(off):
                ts, _ = combine([entry_dot(off + j) for j in range(L)], 1)
                out_v[e, h, pl.ds(off, L)] = ts + biasg_v[slot, pl.ds(off, L)]

            def gbody(g, carry):
                group(g * L)
                return carry

            lax.fori_loop(0, CH // L, gbody, 0)

            NT = CH - (CH // L) * L
            if NT:
                v, d = combine([entry_dot(CH - NT + j) for j in range(NT)], 1)
                while d < L:
                    v = v + _permute(v, lanes ^ d)
                    d *= 2
                prev = out_v[e, h, pl.ds(CH - L, L)]
                out_v[e, h, pl.ds(CH - L, L)] = jnp.where(
                    lanes < L - NT, prev,
                    v + biasg_v[slot, pl.ds(CH - L, L)])

        for b in range(NB):
            start(b, b)

        def outer(g, carry):
            for b in range(NB):
                c = NB * g + b
                rows_copy(c, b).wait()
                bias_copy(c, b).wait()
                compute(c, b)

                @pl.when(c + NB < CPW)
                def _():
                    start(c + NB, b)
            return carry

        lax.fori_loop(0, CPW // NB, outer, 0)

        pltpu.sync_copy(out_v, out_hbm.at[pl.ds(wid * EPW, EPW)])

    return run(embed, idx2d, weight, bias1d)


def kernel(embed, shortlist, weight, bias):
    B, D = embed.shape
    S = shortlist.shape[1]
    idx2d = shortlist.astype(jnp.int32).reshape(B * S // CH, CH)
    bias1d = bias.T.reshape(bias.shape[0])
    return _sc_gemv(embed, idx2d, weight, bias1d, B, S, D).reshape(B, S)
